# Initial kernel scaffold; baseline (speedup 1.0000x reference)
import jax
import jax.numpy as jnp
from jax.experimental import pallas as pl


def kernel(input_ids, att_mask, word_emb, pos_emb, type_emb, emb_ln_g, emb_ln_b, wqkv, bqkv, wo, bo, w1, b1, w2, b2, ln1_g, ln1_b, ln2_g, ln2_b):
    raise NotImplementedError("write your pallas kernel here")


# speedup vs baseline: 13.9262x; 13.9262x over previous
import math

import jax
import jax.numpy as jnp
from jax.experimental import pallas as pl
from jax.experimental.pallas import tpu as pltpu

_VOCAB = 101
_HIDDEN = 32
_NUM_HEADS = 2
_HEAD_DIM = _HIDDEN // _NUM_HEADS
_FFN = 64
_NUM_LAYERS = 2
_LN_EPS = 1e-12
_SEQ = 16
_VPAD = 128


def _mxu_dot(a, b):
    return jnp.dot(a.astype(jnp.bfloat16), b.astype(jnp.bfloat16),
                   preferred_element_type=jnp.float32)


def _layernorm(x, g, b):
    mu = jnp.mean(x, axis=-1, keepdims=True)
    xc = x - mu
    var = jnp.mean(xc * xc, axis=-1, keepdims=True)
    return xc * jax.lax.rsqrt(var + _LN_EPS) * g + b


def _fused_kernel(ids_ref, mask_pool_ref, mask_keys_ref,
                  wemb_hi_ref, wemb_lo_ref, posplus_ref, eg_ref, eb_ref,
                  wqkv_ref, bqkv_ref, wo_ref, bo_ref,
                  w1_ref, b1_ref, w2_ref, b2_ref,
                  ln1g_ref, ln1b_ref, ln2g_ref, ln2b_ref,
                  out_ref):
    R, S = ids_ref.shape
    T = R * S
    G = R // 8
    scale = 1.0 / math.sqrt(_HEAD_DIM)

    ids = ids_ref[...]
    hot = (ids[:, :, None] ==
           jax.lax.broadcasted_iota(jnp.int32, (R, S, _VPAD), 2))
    oh = jnp.where(hot, 1.0, 0.0).astype(jnp.bfloat16).reshape(T, _VPAD)
    emb = (jnp.dot(oh, wemb_hi_ref[...], preferred_element_type=jnp.float32)
           + jnp.dot(oh, wemb_lo_ref[...], preferred_element_type=jnp.float32))
    emb = (emb.reshape(R, S, _HIDDEN) + posplus_ref[...][None]).reshape(T, _HIDDEN)
    x = _layernorm(emb, eg_ref[...], eb_ref[...])

    qseq = jax.lax.broadcasted_iota(jnp.int32, (128, 128), 0) // _SEQ
    kseq = jax.lax.broadcasted_iota(jnp.int32, (128, 128), 1) // _SEQ
    same = (qseq == kseq)[None]
    mkf = mask_keys_ref[...].astype(jnp.float32)
    bias = jnp.where(same, (1.0 - mkf)[:, None, :] * (-1e9), -2e9)

    for l in range(_NUM_LAYERS):
        acc = jnp.zeros((T, _HIDDEN), jnp.float32)
        for h in range(_NUM_HEADS):
            q = _mxu_dot(x, wqkv_ref[l, h]) + bqkv_ref[l, h]
            k = _mxu_dot(x, wqkv_ref[l, _NUM_HEADS + h]) + bqkv_ref[l, _NUM_HEADS + h]
            v = _mxu_dot(x, wqkv_ref[l, 2 * _NUM_HEADS + h]) + bqkv_ref[l, 2 * _NUM_HEADS + h]
            qg = q.reshape(G, 128, _HEAD_DIM)
            kg = k.reshape(G, 128, _HEAD_DIM)
            vg = v.reshape(G, 128, _HEAD_DIM)
            s = jax.lax.dot_general(
                qg, kg, (((2,), (2,)), ((0,), (0,))),
                preferred_element_type=jnp.float32) * scale + bias
            s = s - jnp.max(s, axis=-1, keepdims=True)
            p = jnp.exp(s)
            p = p * pl.reciprocal(jnp.sum(p, axis=-1, keepdims=True), approx=True)
            ctx = jax.lax.dot_general(
                p, vg, (((2,), (1,)), ((0,), (0,))),
                preferred_element_type=jnp.float32)
            acc = acc + _mxu_dot(ctx.reshape(T, _HEAD_DIM), wo_ref[l, h])
        x1 = _layernorm(acc + bo_ref[l] + x, ln1g_ref[l], ln1b_ref[l])

        ff = _mxu_dot(x1, w1_ref[l]) + b1_ref[l]
        ff = jax.nn.gelu(ff, approximate=True)
        ff = _mxu_dot(ff, w2_ref[l]) + b2_ref[l]
        x = _layernorm(ff + x1, ln2g_ref[l], ln2b_ref[l])

    x3 = x.reshape(R, S, _HIDDEN)
    mpf = mask_pool_ref[...].astype(jnp.float32)[:, :, None]
    summed = jnp.sum(x3 * mpf, axis=1)
    counts = jnp.sum(mpf, axis=1)
    pooled = summed / jnp.maximum(counts, 1e-9)
    sq = jnp.sum(pooled * pooled, axis=-1, keepdims=True)
    e = pooled * jax.lax.rsqrt(jnp.maximum(sq, 1e-24))

    e3 = e.reshape(G, 8, _HIDDEN)
    sc = jnp.sum(e3 * e3[:, 0:1, :], axis=-1)
    prob = 1.0 / (1.0 + jnp.exp(-sc))
    out_ref[...] = jnp.clip(prob, 1e-8, 1.0 - 1e-8)


def kernel(input_ids, att_mask, word_emb, pos_emb, type_emb, emb_ln_g, emb_ln_b,
           wqkv, bqkv, wo, bo, w1, b1, w2, b2, ln1_g, ln1_b, ln2_g, ln2_b):
    batch_size, num_samples, seq = input_ids.shape
    num_neg = num_samples - 2
    rows = batch_size * num_samples

    ids2 = input_ids.reshape(rows, seq)
    mask_pool = att_mask.reshape(rows, seq)
    mask_keys = att_mask.reshape(rows // 8, 8 * seq)

    wpad = jnp.zeros((_VPAD, _HIDDEN), jnp.float32).at[:_VOCAB].set(word_emb)
    wemb_hi = wpad.astype(jnp.bfloat16)
    wemb_lo = (wpad - wemb_hi.astype(jnp.float32)).astype(jnp.bfloat16)
    posplus = pos_emb[:seq] + type_emb[0][None, :]

    R = 256 if rows % 256 == 0 else 8
    grid = (rows // R,)
    G = R // 8

    def batched(shape2plus):
        nz = len(shape2plus) - 1
        return pl.BlockSpec(shape2plus, lambda i, nz=nz: (i,) + (0,) * nz)

    def full(arr):
        rank = arr.ndim
        return pl.BlockSpec(arr.shape, lambda i, rank=rank: (0,) * rank)

    consts = (wemb_hi, wemb_lo, posplus, emb_ln_g, emb_ln_b,
              wqkv, bqkv, wo, bo, w1, b1, w2, b2, ln1_g, ln1_b, ln2_g, ln2_b)

    out = pl.pallas_call(
        _fused_kernel,
        out_shape=jax.ShapeDtypeStruct((rows // 8, 8), jnp.float32),
        grid=grid,
        in_specs=([batched((R, seq)), batched((R, seq)), batched((G, 8 * seq))]
                  + [full(a) for a in consts]),
        out_specs=pl.BlockSpec((G, 8), lambda i: (i, 0)),
        compiler_params=pltpu.CompilerParams(
            dimension_semantics=("parallel",),
            vmem_limit_bytes=64 * 1024 * 1024),
    )(ids2, mask_pool, mask_keys, *consts)

    pos_out = out[:, 1]
    neg_out = out[:, 2:2 + num_neg]
    return pos_out, neg_out
